# baseline (speedup 1.0000x reference)
import functools

import jax
import jax.numpy as jnp
from jax import lax
from jax.experimental import pallas as pl
from jax.experimental.pallas import tpu as pltpu
from jax.experimental.pallas import tpu_sc as plsc

_NC = 2
_NS = 16
_NW = _NC * _NS

_T = 16384
_D = 32
_LATENT = 128

_CH = 128
_B_PER_W = _T // _NW
_NCH = _B_PER_W // _CH


def _make_sc_gather():
  mesh = plsc.VectorSubcoreMesh(
      core_axis_name="c", subcore_axis_name="s",
      num_cores=_NC, num_subcores=_NS)

  @functools.partial(
      pl.kernel,
      out_type=jax.ShapeDtypeStruct((_NW, _NCH, _CH, _D), jnp.float32),
      mesh=mesh,
      compiler_params=pltpu.CompilerParams(use_tc_tiling_on_sc=False),
      scratch_types=[
          pltpu.VMEM((_NCH, _CH), jnp.int32),
          pltpu.VMEM((_NCH, _CH, _D), jnp.float32),
          pltpu.SemaphoreType.DMA,
      ],
  )
  def gather_kernel(table_hbm, idx_hbm, out_hbm, idx_v, rows_v, sem):
    wid = lax.axis_index("s") * _NC + lax.axis_index("c")
    pltpu.sync_copy(idx_hbm.at[wid], idx_v)
    copies = []
    for j in range(_NCH):
      copies.append(
          pltpu.async_copy(table_hbm.at[idx_v.at[j]], rows_v.at[j], sem))
    for c in copies:
      c.wait()
    pltpu.sync_copy(rows_v, out_hbm.at[wid])

  return gather_kernel


_sc_gather = _make_sc_gather()


def _proj_body(emb_ref, wt_ref, b_ref, out_ref):
  out_ref[...] = jnp.dot(
      emb_ref[...], wt_ref[...],
      preferred_element_type=jnp.float32) + b_ref[...]


def _tc_project(emb, wt, b2d):
  bt = 2048
  grid = (_T // bt,)
  return pl.pallas_call(
      _proj_body,
      grid=grid,
      in_specs=[
          pl.BlockSpec((bt, _D), lambda i: (i, 0)),
          pl.BlockSpec((_D, _LATENT), lambda i: (0, 0)),
          pl.BlockSpec((1, _LATENT), lambda i: (0, 0)),
      ],
      out_specs=pl.BlockSpec((bt, _LATENT), lambda i: (i, 0)),
      out_shape=jax.ShapeDtypeStruct((_T, _LATENT), jnp.float32),
  )(emb, wt, b2d)


def kernel(idx, embed, W, b):
  idx3 = idx.astype(jnp.int32).reshape(_NW, _NCH, _CH)
  emb = _sc_gather(embed, idx3).reshape(_T, _D)
  return _tc_project(emb, W.T, b.reshape(1, _LATENT))


# speedup vs baseline: 1.6260x; 1.6260x over previous
import functools

import jax
import jax.numpy as jnp
from jax import lax
from jax.experimental import pallas as pl
from jax.experimental.pallas import tpu as pltpu
from jax.experimental.pallas import tpu_sc as plsc

_NC = 2
_NS = 16
_NW = _NC * _NS

_T = 16384
_D = 32
_V = 1000002
_LATENT = 128

_SLAB = 512
_NSLAB = (_V + _SLAB - 1) // _SLAB
_TAIL_ID = _NSLAB - 1
_TAIL_W = _V - _TAIL_ID * _SLAB
_KMAX = (_NSLAB + _NW - 1) // _NW
_CAP = _T // 16
_NGRP = _T // 16
_OUT_RING = 4


def _make_sc_gather():
  mesh = plsc.VectorSubcoreMesh(
      core_axis_name="c", subcore_axis_name="s",
      num_cores=_NC, num_subcores=_NS)

  @functools.partial(
      pl.kernel,
      out_type=jax.ShapeDtypeStruct((_T + 16, _LATENT), jnp.float32),
      mesh=mesh,
      compiler_params=pltpu.CompilerParams(
          use_tc_tiling_on_sc=True, needs_layout_passes=False),
      scratch_types=[
          pltpu.VMEM((_T,), jnp.int32),
          pltpu.VMEM((_T,), jnp.int32),
          pltpu.VMEM((_T,), jnp.int32),
          pltpu.VMEM((_KMAX + 1, 16), jnp.int32),
          pltpu.VMEM((_KMAX, 16), jnp.int32),
          pltpu.VMEM((_KMAX, 16), jnp.int32),
          pltpu.VMEM((2, _D, _SLAB), jnp.float32),
          pltpu.VMEM((_D, _TAIL_W), jnp.float32),
          pltpu.VMEM((_OUT_RING, 16, _LATENT), jnp.float32),
          pltpu.SemaphoreType.DMA,
          pltpu.SemaphoreType.DMA,
      ],
  )
  def gather_kernel(tablet_hbm, tail_hbm, idx_hbm, out_hbm,
                    idx_v, listv, listt, offs, cnt, curs,
                    slab_ring, tail_buf, stage_ring, sem_in, sem_out):
    wid = lax.axis_index("s") * _NC + lax.axis_index("c")
    lanes = lax.iota(jnp.int32, 16)
    zeros16 = jnp.zeros((16,), jnp.int32)
    wid_splat = zeros16 + wid

    pltpu.sync_copy(idx_hbm, idx_v)

    def zero_cnt(k, c):
      cnt[k, :] = zeros16
      return c
    lax.fori_loop(0, _KMAX, zero_cnt, 0)
    fzeros = jnp.zeros((16,), jnp.float32)
    for slot in range(_OUT_RING):
      for r in range(16):
        for c0 in range(_D, _LATENT, 16):
          stage_ring[slot, r, pl.ds(c0, 16)] = fzeros

    def hist(g, c):
      v = idx_v[pl.ds(g * 16, 16)]
      sg = v // _SLAB
      m = (sg - (sg // _NW) * _NW) == wid_splat
      kloc = sg // _NW
      plsc.addupdate_scatter(cnt, [kloc, lanes], zeros16 + 1, mask=m)
      return c
    lax.fori_loop(0, _NGRP, hist, 0, unroll=4)

    def scan(k, running):
      offs[k, :] = running
      curs[k, :] = running
      return running + cnt[k, :]
    total = lax.fori_loop(0, _KMAX, scan, zeros16)
    offs[_KMAX, :] = total

    def place(g, c):
      v = idx_v[pl.ds(g * 16, 16)]
      sg = v // _SLAB
      m = (sg - (sg // _NW) * _NW) == wid_splat
      kloc = sg // _NW
      pos = plsc.load_gather(curs, [kloc, lanes], mask=m)
      li = lanes * _CAP + pos
      plsc.store_scatter(listv, [li], v, mask=m)
      plsc.store_scatter(listt, [li], zeros16 + g * 16 + lanes, mask=m)
      plsc.store_scatter(curs, [kloc, lanes], pos + 1, mask=m)
      return c
    lax.fori_loop(0, _NGRP, place, 0, unroll=4)

    def fire_slab(k, par):
      sid = k * _NW + wid
      start = pl.multiple_of(sid * _SLAB, _SLAB)

      @pl.when(sid < _TAIL_ID)
      def _():
        pltpu.async_copy(
            tablet_hbm.at[:, pl.ds(start, _SLAB)], slab_ring.at[par], sem_in)

      @pl.when(sid == _TAIL_ID)
      def _():
        pltpu.async_copy(tail_hbm, tail_buf, sem_in)

    def wait_slab(k, par):
      sid = k * _NW + wid

      @pl.when(sid < _TAIL_ID)
      def _():
        pltpu.make_async_copy(
            tablet_hbm.at[:, pl.ds(0, _SLAB)], slab_ring.at[par],
            sem_in).wait()

      @pl.when(sid == _TAIL_ID)
      def _():
        pltpu.make_async_copy(tail_hbm, tail_buf, sem_in).wait()
        for e in range(_D):
          esplat = jnp.zeros((16,), jnp.int32) + e
          for c0 in range(0, _TAIL_W + 15, 16):
            cols = lax.iota(jnp.int32, 16) + c0
            m = cols < _TAIL_W
            w_e = plsc.load_gather(tail_buf, [esplat, cols], mask=m)
            plsc.store_scatter(slab_ring.at[par], [esplat, cols], w_e, mask=m)

    fire_slab(0, 0)

    def slab_body(k, scnt):
      par = lax.rem(k, 2)
      sid = k * _NW + wid
      wait_slab(k, par)

      @pl.when((k + 1) * _NW + wid < _NSLAB)
      def _():
        fire_slab(k + 1, 1 - par)

      st = offs[k, :]
      en = offs[k + 1, :]
      rmax = lax.reduce_max(en - st, (0,))
      base_col = sid * _SLAB

      def rank_body(r, sc):
        valid = (zeros16 + r) < (en - st)
        li = lanes * _CAP + st + r
        lv = plsc.load_gather(listv, [li], mask=valid)
        lt = plsc.load_gather(listt, [li], mask=valid)
        col = lv - base_col
        slot = lax.rem(sc, _OUT_RING)

        @pl.when(sc >= _OUT_RING)
        def _():
          pltpu.make_async_copy(
              stage_ring.at[0], out_hbm.at[_T + lanes], sem_out).wait()

        stg = stage_ring.at[slot]
        for e in range(_D):
          esplat = zeros16 + e
          w_e = plsc.load_gather(slab_ring.at[par], [esplat, col], mask=valid)
          plsc.store_scatter(stg, [lanes, esplat], w_e, mask=valid)
        ids = jnp.where(valid, lt, _T + lanes)
        pltpu.async_copy(stg, out_hbm.at[ids], sem_out)
        return sc + 1

      return lax.fori_loop(0, rmax, rank_body, scnt)

    scnt = lax.fori_loop(0, _KMAX, slab_body, 0)

    for i in range(_OUT_RING):
      @pl.when(i < scnt)
      def _():
        pltpu.make_async_copy(
            stage_ring.at[0], out_hbm.at[_T + lanes], sem_out).wait()

  return gather_kernel


_sc_gather = _make_sc_gather()


def _proj_body(emb_ref, wt_ref, b_ref, out_ref):
  out_ref[...] = jnp.dot(
      emb_ref[...], wt_ref[...],
      preferred_element_type=jnp.float32) + b_ref[...]


def _tc_project(emb, wt, b2d):
  bt = 2048
  grid = (_T // bt,)
  return pl.pallas_call(
      _proj_body,
      grid=grid,
      in_specs=[
          pl.BlockSpec((bt, _LATENT), lambda i: (i, 0)),
          pl.BlockSpec((_LATENT, _LATENT), lambda i: (0, 0)),
          pl.BlockSpec((1, _LATENT), lambda i: (0, 0)),
      ],
      out_specs=pl.BlockSpec((bt, _LATENT), lambda i: (i, 0)),
      out_shape=jax.ShapeDtypeStruct((_T, _LATENT), jnp.float32),
  )(emb, wt, b2d)


def kernel(idx, embed, W, b):
  tablet = embed.T
  emb128 = _sc_gather(tablet, tablet[:, _TAIL_ID * _SLAB:],
                      idx.astype(jnp.int32))
  wt_pad = jnp.zeros((_LATENT, _LATENT), jnp.float32).at[:_D].set(W.T)
  return _tc_project(emb128, wt_pad, b.reshape(1, _LATENT))
